# baseline (speedup 1.0000x reference)
import jax
import jax.numpy as jnp
from jax.experimental import pallas as pl
from jax.experimental.pallas import tpu as pltpu

_B, _L, _H, _C = 16, 4096, 512, 2
_CHUNK = 512
_NCH = _L // _CHUNK


def _lengths_kernel(m1_ref, m2_ref, len_ref):
    for i, m_ref in enumerate((m1_ref, m2_ref)):
        m = m_ref[...]
        mn = jnp.min(m, axis=1, keepdims=True)
        pos = jax.lax.broadcasted_iota(jnp.int32, m.shape, 1)
        am = jnp.min(jnp.where(m == mn, pos, _L), axis=1)
        len_ref[i, :] = jnp.where(am == 0, _L, am)


def _pool_kernel(len_ref, x1_ref, x2_ref, w_ref, bias_ref, out_ref, acc1, acc2):
    b = pl.program_id(0)
    c = pl.program_id(1)
    neg = jnp.finfo(jnp.float32).min

    for i, (x_ref, acc) in enumerate(((x1_ref, acc1), (x2_ref, acc2))):
        eff = len_ref[i, b]
        nch = pl.cdiv(eff, _CHUNK)

        @pl.when(c == 0)
        def _():
            acc[b, :] = jnp.full((_H,), neg, jnp.float32)

        @pl.when(c < nch)
        def _():
            x = x_ref[0]
            pos = c * _CHUNK + jax.lax.broadcasted_iota(jnp.int32, x.shape, 0)
            xm = jnp.where(pos < eff, x, neg)
            acc[b, :] = jnp.maximum(acc[b, :], jnp.max(xm, axis=0))

    @pl.when((b == _B - 1) & (c == _NCH - 1))
    def _():
        h1 = acc1[...]
        h2 = acc2[...]
        out = (
            jnp.dot(h1, w_ref[:_H, :], preferred_element_type=jnp.float32)
            + jnp.dot(h2, w_ref[_H:, :], preferred_element_type=jnp.float32)
            + bias_ref[0, :]
        )
        out_ref[...] = out


def kernel(x1, x2, m1, m2, W, b):
    lens = pl.pallas_call(
        _lengths_kernel,
        out_shape=jax.ShapeDtypeStruct((2, _B), jnp.int32),
    )(m1, m2)

    grid_spec = pltpu.PrefetchScalarGridSpec(
        num_scalar_prefetch=1,
        grid=(_B, _NCH),
        in_specs=[
            pl.BlockSpec(
                (1, _CHUNK, _H),
                lambda bb, cc, lens: (bb, jnp.minimum(cc, pl.cdiv(lens[0, bb], _CHUNK) - 1), 0),
            ),
            pl.BlockSpec(
                (1, _CHUNK, _H),
                lambda bb, cc, lens: (bb, jnp.minimum(cc, pl.cdiv(lens[1, bb], _CHUNK) - 1), 0),
            ),
            pl.BlockSpec((2 * _H, _C), lambda bb, cc, lens: (0, 0)),
            pl.BlockSpec((1, _C), lambda bb, cc, lens: (0, 0)),
        ],
        out_specs=pl.BlockSpec((_B, _C), lambda bb, cc, lens: (0, 0)),
        scratch_shapes=[
            pltpu.VMEM((_B, _H), jnp.float32),
            pltpu.VMEM((_B, _H), jnp.float32),
        ],
    )

    out = pl.pallas_call(
        _pool_kernel,
        grid_spec=grid_spec,
        out_shape=jax.ShapeDtypeStruct((_B, _C), jnp.float32),
    )(lens, x1, x2, W, b.reshape(1, _C))
    return out


# speedup vs baseline: 1.0023x; 1.0023x over previous
import jax
import jax.numpy as jnp
from jax.experimental import pallas as pl
from jax.experimental.pallas import tpu as pltpu

_B, _L, _H, _C = 16, 4096, 512, 2
_CHUNK = 512
_NCH = _L // _CHUNK


def _lengths_kernel(m1_ref, m2_ref, len_ref):
    for i, m_ref in enumerate((m1_ref, m2_ref)):
        m = m_ref[...]
        mn = jnp.min(m, axis=1, keepdims=True)
        pos = jax.lax.broadcasted_iota(jnp.int32, m.shape, 1)
        am = jnp.min(jnp.where(m == mn, pos, _L), axis=1)
        len_ref[i, :] = jnp.where(am == 0, _L, am)


def _pool_kernel(len_ref, x1_ref, x2_ref, w_ref, bias_ref, out_ref, acc1, acc2):
    b = pl.program_id(0)
    c = pl.program_id(1)
    neg = jnp.finfo(jnp.float32).min

    for i, (x_ref, acc) in enumerate(((x1_ref, acc1), (x2_ref, acc2))):
        eff = len_ref[i, b]
        nch = pl.cdiv(eff, _CHUNK)

        @pl.when(c == 0)
        def _():
            acc[b] = jnp.full((8, _H), neg, jnp.float32)

        @pl.when(c + 1 < nch)
        def _():
            x = x_ref[0].reshape(_CHUNK // 8, 8, _H)
            acc[b] = jnp.maximum(acc[b], jnp.max(x, axis=0))

        @pl.when(c + 1 == nch)
        def _():
            x = x_ref[0]
            pos = c * _CHUNK + jax.lax.broadcasted_iota(jnp.int32, x.shape, 0)
            xm = jnp.where(pos < eff, x, neg).reshape(_CHUNK // 8, 8, _H)
            acc[b] = jnp.maximum(acc[b], jnp.max(xm, axis=0))

    @pl.when((b == _B - 1) & (c == _NCH - 1))
    def _():
        h1 = jnp.max(acc1[...], axis=1)
        h2 = jnp.max(acc2[...], axis=1)
        out = (
            jnp.dot(h1, w_ref[:_H, :], preferred_element_type=jnp.float32)
            + jnp.dot(h2, w_ref[_H:, :], preferred_element_type=jnp.float32)
            + bias_ref[0, :]
        )
        out_ref[...] = out


def kernel(x1, x2, m1, m2, W, b):
    lens = pl.pallas_call(
        _lengths_kernel,
        out_shape=jax.ShapeDtypeStruct((2, _B), jnp.int32),
    )(m1, m2)

    grid_spec = pltpu.PrefetchScalarGridSpec(
        num_scalar_prefetch=1,
        grid=(_B, _NCH),
        in_specs=[
            pl.BlockSpec(
                (1, _CHUNK, _H),
                lambda bb, cc, lens: (bb, jnp.minimum(cc, pl.cdiv(lens[0, bb], _CHUNK) - 1), 0),
            ),
            pl.BlockSpec(
                (1, _CHUNK, _H),
                lambda bb, cc, lens: (bb, jnp.minimum(cc, pl.cdiv(lens[1, bb], _CHUNK) - 1), 0),
            ),
            pl.BlockSpec((2 * _H, _C), lambda bb, cc, lens: (0, 0)),
            pl.BlockSpec((1, _C), lambda bb, cc, lens: (0, 0)),
        ],
        out_specs=pl.BlockSpec((_B, _C), lambda bb, cc, lens: (0, 0)),
        scratch_shapes=[
            pltpu.VMEM((_B, 8, _H), jnp.float32),
            pltpu.VMEM((_B, 8, _H), jnp.float32),
        ],
    )

    out = pl.pallas_call(
        _pool_kernel,
        grid_spec=grid_spec,
        out_shape=jax.ShapeDtypeStruct((_B, _C), jnp.float32),
    )(lens, x1, x2, W, b.reshape(1, _C))
    return out


# speedup vs baseline: 1.7950x; 1.7908x over previous
import jax
import jax.numpy as jnp
from jax.experimental import pallas as pl
from jax.experimental.pallas import tpu as pltpu

_B, _L, _H, _C = 16, 4096, 512, 2
_CHUNK = 512
_NCH = _L // _CHUNK


def _lengths_kernel(m1_ref, m2_ref, len_ref):
    for i, m_ref in enumerate((m1_ref, m2_ref)):
        m = m_ref[...]
        mn = jnp.min(m, axis=1, keepdims=True)
        pos = jax.lax.broadcasted_iota(jnp.int32, m.shape, 1)
        am = jnp.min(jnp.where(m == mn, pos, _L), axis=1)
        len_ref[i, :] = jnp.where(am == 0, _L, am)


def _pool_kernel(len_ref, x1_hbm, x2_hbm, w_ref, bias_ref, out_ref, acc1, acc2):
    neg = jnp.finfo(jnp.float32).min
    acc1[...] = jnp.full(acc1.shape, neg, jnp.float32)
    acc2[...] = jnp.full(acc2.shape, neg, jnp.float32)

    def inner(idx, x1_blk, x2_blk):
        b, c = idx
        for i, (blk, acc) in enumerate(((x1_blk, acc1), (x2_blk, acc2))):
            eff = len_ref[i, b]
            nch = pl.cdiv(eff, _CHUNK)

            @pl.when(c + 1 < nch)
            def _():
                x = blk[0].reshape(_CHUNK // 8, 8, _H)
                acc[b] = jnp.maximum(acc[b], jnp.max(x, axis=0))

            @pl.when(c + 1 == nch)
            def _():
                x = blk[0]
                pos = c * _CHUNK + jax.lax.broadcasted_iota(jnp.int32, x.shape, 0)
                xm = jnp.where(pos < eff, x, neg).reshape(_CHUNK // 8, 8, _H)
                acc[b] = jnp.maximum(acc[b], jnp.max(xm, axis=0))

    def _mk_index_map(i):
        def index_map(b, c):
            nch = pl.cdiv(len_ref[i, b], _CHUNK)
            return (b, jnp.minimum(c, nch - 1), 0)
        return index_map

    pipe = pltpu.emit_pipeline(
        inner,
        grid=(_B, _NCH),
        in_specs=[
            pl.BlockSpec(
                (1, _CHUNK, _H),
                _mk_index_map(0),
                pipeline_mode=pl.Buffered(buffer_count=8, use_lookahead=True),
            ),
            pl.BlockSpec(
                (1, _CHUNK, _H),
                _mk_index_map(1),
                pipeline_mode=pl.Buffered(buffer_count=8, use_lookahead=True),
            ),
        ],
        _explicit_indices=True,
    )
    pipe(x1_hbm, x2_hbm)

    h1 = jnp.max(acc1[...], axis=1)
    h2 = jnp.max(acc2[...], axis=1)
    out_ref[...] = (
        jnp.dot(h1, w_ref[:_H, :], preferred_element_type=jnp.float32)
        + jnp.dot(h2, w_ref[_H:, :], preferred_element_type=jnp.float32)
        + bias_ref[0, :]
    )


def kernel(x1, x2, m1, m2, W, b):
    lens = pl.pallas_call(
        _lengths_kernel,
        out_shape=jax.ShapeDtypeStruct((2, _B), jnp.int32),
    )(m1, m2)

    out = pl.pallas_call(
        _pool_kernel,
        in_specs=[
            pl.BlockSpec(memory_space=pltpu.SMEM),
            pl.BlockSpec(memory_space=pl.ANY),
            pl.BlockSpec(memory_space=pl.ANY),
            pl.BlockSpec(memory_space=pltpu.VMEM),
            pl.BlockSpec(memory_space=pltpu.VMEM),
        ],
        out_specs=pl.BlockSpec(memory_space=pltpu.VMEM),
        out_shape=jax.ShapeDtypeStruct((_B, _C), jnp.float32),
        scratch_shapes=[
            pltpu.VMEM((_B, 8, _H), jnp.float32),
            pltpu.VMEM((_B, 8, _H), jnp.float32),
        ],
    )(lens, x1, x2, W, b.reshape(1, _C))
    return out
